# baseline (speedup 1.0000x reference)
import functools

import jax
import jax.numpy as jnp
from jax import lax
from jax.experimental import pallas as pl
from jax.experimental.pallas import tpu as pltpu
from jax.experimental.pallas import tpu_sc as plsc

T = 160000
R = 10000
NB = 4
D = 128
NW = 32
CHUNK = T // NW
NV = CHUNK // 16
TAIL = CHUNK - NV * 16
QCAP = 2 * CHUNK + 16
CG = (2 * R * NB) // D
TW = CHUNK * 3
TVW = (NV + 1) * 48

_mesh = plsc.VectorSubcoreMesh(core_axis_name="c", subcore_axis_name="s")


@functools.partial(
    pl.kernel,
    mesh=_mesh,
    compiler_params=pltpu.CompilerParams(needs_layout_passes=False),
    out_type=[
        jax.ShapeDtypeStruct((NW, NB, 2 * D), jnp.float32),
        jax.ShapeDtypeStruct((NW, 16), jnp.float32),
    ],
    scratch_types=[
        pltpu.VMEM((TVW,), jnp.int32),
        pltpu.VMEM((16,), jnp.int32),
        pltpu.VMEM((QCAP,), jnp.int32),
        pltpu.VMEM((QCAP,), jnp.int32),
        pltpu.VMEM((QCAP,), jnp.int32),
        pltpu.VMEM((16,), jnp.int32),
        pltpu.VMEM((NB, 2 * D), jnp.float32),
        pltpu.VMEM((16, D), jnp.float32),
        pltpu.VMEM((16, D), jnp.float32),
        pltpu.VMEM((16, D), jnp.float32),
        pltpu.VMEM((16,), jnp.float32),
        pltpu.SemaphoreType.DMA,
    ],
)
def _sc_scan(tri_hbm, u_hbm, ent_hbm, rel_hbm, comp_hbm,
             part_out, cnt_out,
             tri_v, u_v, qe_v, qr_v, qc_v, qg_v, s_acc, ebuf, rbuf,
             cbuf, cntf_v, sem):
    wid = lax.axis_index("s") * 2 + lax.axis_index("c")
    pltpu.sync_copy(tri_hbm.at[pl.ds(wid * TW, TW)], tri_v.at[pl.ds(0, TW)])
    pltpu.sync_copy(u_hbm, u_v)

    zeros16 = jnp.zeros((16,), jnp.float32)
    for b in range(NB):
        for k in range(2 * D // 16):
            s_acc[b, pl.ds(k * 16, 16)] = zeros16

    lane16 = lax.iota(jnp.int32, 16)
    iota3 = lane16 * jnp.full((16,), 3, jnp.int32)
    ones16 = jnp.full((16,), 1, jnp.int32)

    def scan_step(i, cnt, lane_valid):
        idx0 = jnp.full((16,), i * 48, jnp.int32) + iota3
        sv = plsc.load_gather(tri_v, [idx0])
        rv = plsc.load_gather(tri_v, [idx0 + ones16])
        dv = plsc.load_gather(tri_v, [idx0 + ones16 + ones16])
        uv = u_v[...]
        mf = dv == uv
        mr = sv == uv
        if lane_valid is not None:
            mf = jnp.logical_and(mf, lane_valid)
            mr = jnp.logical_and(mr, lane_valid)
        nt = plsc.all_reduce_population_count(jnp.logical_or(mf, mr))[0]

        @pl.when(nt > 0)
        def _():
            nf = plsc.all_reduce_population_count(mf)[0]
            plsc.store_compressed(qe_v.at[pl.ds(cnt, 16)], sv, mask=mf)
            plsc.store_compressed(qr_v.at[pl.ds(cnt, 16)], rv, mask=mf)
            plsc.store_compressed(qc_v.at[pl.ds(cnt, 16)], rv, mask=mf)
            c2 = cnt + nf
            plsc.store_compressed(qe_v.at[pl.ds(c2, 16)], dv, mask=mr)
            plsc.store_compressed(qr_v.at[pl.ds(c2, 16)], rv, mask=mr)
            plsc.store_compressed(
                qc_v.at[pl.ds(c2, 16)],
                rv + jnp.full((16,), R, jnp.int32), mask=mr)

        return cnt + nt

    n = lax.fori_loop(0, NV, lambda i, c: scan_step(i, c, None), jnp.int32(0))
    n = scan_step(NV, n, lane16 < jnp.full((16,), TAIL, jnp.int32))

    nb = (n + 15) // 16

    @pl.when(n > 0)
    def _():
        toff = (nb - 1) * 16
        valid_tail = (lane16 + jnp.full((16,), toff, jnp.int32)) < jnp.full(
            (16,), n, jnp.int32)
        for q in (qe_v, qr_v, qc_v):
            qv = q[pl.ds(toff, 16)]
            q[pl.ds(toff, 16)] = jnp.where(valid_tail, qv,
                                           jnp.zeros((16,), jnp.int32))

    def batch_body(j, _):
        qoff = j * 16
        qrow = qc_v[pl.ds(qoff, 16)]
        qg_v[...] = lax.shift_right_logical(qrow, 5)
        cp_e = pltpu.async_copy(ent_hbm.at[qe_v.at[pl.ds(qoff, 16)]], ebuf,
                                sem)
        cp_r = pltpu.async_copy(rel_hbm.at[qr_v.at[pl.ds(qoff, 16)]], rbuf,
                                sem)
        cp_c = pltpu.async_copy(comp_hbm.at[qg_v], cbuf, sem)
        cp_e.wait()
        cp_r.wait()
        cp_c.wait()
        colb = lax.shift_left(
            jnp.bitwise_and(qrow, jnp.full((16,), 31, jnp.int32)),
            jnp.full((16,), 2, jnp.int32))

        for e in range(16):
            gvalid = (qoff + e) < n
            cols = jnp.full((16,), colb[e], jnp.int32) + lane16
            rows = jnp.full((16,), e, jnp.int32)
            crow = plsc.load_gather(cbuf, [rows, cols])
            cbs = [jnp.where(gvalid, crow[b], 0.0) for b in range(NB)]
            for k in range(D // 16):
                ev = ebuf[e, pl.ds(k * 16, 16)]
                rv2 = rbuf[e, pl.ds(k * 16, 16)]
                for b in range(NB):
                    s_acc[b, pl.ds(k * 16, 16)] += cbs[b] * ev
                    s_acc[b, pl.ds(D + k * 16, 16)] += cbs[b] * rv2
        return 0

    lax.fori_loop(0, nb, batch_body, 0)

    pltpu.sync_copy(s_acc, part_out.at[wid])
    cntf_v[...] = jnp.full((16,), n.astype(jnp.float32), jnp.float32)
    pltpu.sync_copy(cntf_v, cnt_out.at[wid])


def _tc_body(part_ref, cnt_ref, bases_ref, out_ref):
    s_total = jnp.sum(part_ref[...], axis=0)
    total = jnp.sum(cnt_ref[...]) * (1.0 / 16.0)
    denom = jnp.maximum(total, 1.0)
    acc = jnp.zeros((1, D), jnp.float32)
    for b in range(NB):
        acc = acc + jnp.dot(s_total[b:b + 1, :], bases_ref[b],
                            preferred_element_type=jnp.float32)
    out_ref[...] = acc / denom


def kernel(unseen_entity, triplets, use_cuda, entity_table, relation_table,
           bases, comp):
    tri_flat = jnp.asarray(triplets).astype(jnp.int32).reshape(T * 3)
    u_arr = jnp.full((16,), jnp.asarray(unseen_entity, jnp.int32))
    comp_g = comp.astype(jnp.float32).reshape(CG, D)

    part, cnt = _sc_scan(tri_flat, u_arr, entity_table, relation_table,
                         comp_g)

    out = pl.pallas_call(
        _tc_body,
        out_shape=jax.ShapeDtypeStruct((1, D), jnp.float32),
    )(part, cnt, bases)
    return out.reshape(D)


# speedup vs baseline: 2.4081x; 2.4081x over previous
import functools

import jax
import jax.numpy as jnp
from jax import lax
from jax.experimental import pallas as pl
from jax.experimental.pallas import tpu as pltpu
from jax.experimental.pallas import tpu_sc as plsc

T = 160000
R = 10000
NB = 4
D = 128
NW = 32
UNROLL = 4
CHUNK = 5056
TPAD = NW * CHUNK
NV = CHUNK // (16 * UNROLL)
QCAP = 2 * CHUNK + 16
CG = (2 * R * NB) // D

_mesh = plsc.VectorSubcoreMesh(core_axis_name="c", subcore_axis_name="s")


@functools.partial(
    pl.kernel,
    mesh=_mesh,
    compiler_params=pltpu.CompilerParams(needs_layout_passes=False),
    out_type=[
        jax.ShapeDtypeStruct((NW, NB, 2 * D), jnp.float32),
        jax.ShapeDtypeStruct((NW, 16), jnp.float32),
    ],
    scratch_types=[
        pltpu.VMEM((CHUNK,), jnp.int32),
        pltpu.VMEM((CHUNK,), jnp.int32),
        pltpu.VMEM((CHUNK,), jnp.int32),
        pltpu.VMEM((16,), jnp.int32),
        pltpu.VMEM((QCAP,), jnp.int32),
        pltpu.VMEM((QCAP,), jnp.int32),
        pltpu.VMEM((QCAP,), jnp.int32),
        pltpu.VMEM((16,), jnp.int32),
        pltpu.VMEM((NB, 2 * D), jnp.float32),
        pltpu.VMEM((16, D), jnp.float32),
        pltpu.VMEM((16, D), jnp.float32),
        pltpu.VMEM((16, D), jnp.float32),
        pltpu.VMEM((16,), jnp.float32),
        pltpu.SemaphoreType.DMA,
    ],
)
def _sc_scan(s_hbm, r_hbm, d_hbm, u_hbm, ent_hbm, rel_hbm, comp_hbm,
             part_out, cnt_out,
             s_v, r_v, d_v, u_v, qe_v, qr_v, qc_v, qg_v, s_acc, ebuf, rbuf,
             cbuf, cntf_v, sem):
    wid = lax.axis_index("s") * 2 + lax.axis_index("c")
    base = wid * CHUNK
    pltpu.sync_copy(s_hbm.at[pl.ds(base, CHUNK)], s_v)
    pltpu.sync_copy(r_hbm.at[pl.ds(base, CHUNK)], r_v)
    pltpu.sync_copy(d_hbm.at[pl.ds(base, CHUNK)], d_v)
    pltpu.sync_copy(u_hbm, u_v)

    zeros16 = jnp.zeros((16,), jnp.float32)
    for b in range(NB):
        for k in range(2 * D // 16):
            s_acc[b, pl.ds(k * 16, 16)] = zeros16

    lane16 = lax.iota(jnp.int32, 16)
    rsplat = jnp.full((16,), R, jnp.int32)

    def scan_body(i, cnt):
        off = i * (16 * UNROLL)
        uv = u_v[...]
        svs = [s_v[pl.ds(off + 16 * h, 16)] for h in range(UNROLL)]
        dvs = [d_v[pl.ds(off + 16 * h, 16)] for h in range(UNROLL)]
        mfs = [dvs[h] == uv for h in range(UNROLL)]
        mrs = [svs[h] == uv for h in range(UNROLL)]
        many = mfs[0]
        for h in range(UNROLL):
            many = jnp.logical_or(many, mfs[h]) if h else many
            many = jnp.logical_or(many, mrs[h])
        nt = plsc.all_reduce_population_count(many)[0]

        @pl.when(nt > 0)
        def _():
            c = cnt
            for h in range(UNROLL):
                rvh = r_v[pl.ds(off + 16 * h, 16)]
                nfh = plsc.all_reduce_population_count(mfs[h])[0]
                nrh = plsc.all_reduce_population_count(mrs[h])[0]
                plsc.store_compressed(qe_v.at[pl.ds(c, 16)], svs[h],
                                      mask=mfs[h])
                plsc.store_compressed(qr_v.at[pl.ds(c, 16)], rvh,
                                      mask=mfs[h])
                plsc.store_compressed(qc_v.at[pl.ds(c, 16)], rvh,
                                      mask=mfs[h])
                c2 = c + nfh
                plsc.store_compressed(qe_v.at[pl.ds(c2, 16)], dvs[h],
                                      mask=mrs[h])
                plsc.store_compressed(qr_v.at[pl.ds(c2, 16)], rvh,
                                      mask=mrs[h])
                plsc.store_compressed(qc_v.at[pl.ds(c2, 16)], rvh + rsplat,
                                      mask=mrs[h])
                c = c2 + nrh

        return cnt + nt

    n = lax.fori_loop(0, NV, scan_body, jnp.int32(0))

    nb = (n + 15) // 16

    @pl.when(n > 0)
    def _():
        toff = (nb - 1) * 16
        valid_tail = (lane16 + jnp.full((16,), toff, jnp.int32)) < jnp.full(
            (16,), n, jnp.int32)
        for q in (qe_v, qr_v, qc_v):
            qv = q[pl.ds(toff, 16)]
            q[pl.ds(toff, 16)] = jnp.where(valid_tail, qv,
                                           jnp.zeros((16,), jnp.int32))

    def batch_body(j, _):
        qoff = j * 16
        qrow = qc_v[pl.ds(qoff, 16)]
        qg_v[...] = lax.shift_right_logical(qrow, 5)
        cp_e = pltpu.async_copy(ent_hbm.at[qe_v.at[pl.ds(qoff, 16)]], ebuf,
                                sem)
        cp_r = pltpu.async_copy(rel_hbm.at[qr_v.at[pl.ds(qoff, 16)]], rbuf,
                                sem)
        cp_c = pltpu.async_copy(comp_hbm.at[qg_v], cbuf, sem)
        cp_e.wait()
        cp_r.wait()
        cp_c.wait()
        colb = lax.shift_left(
            jnp.bitwise_and(qrow, jnp.full((16,), 31, jnp.int32)),
            jnp.full((16,), 2, jnp.int32))

        for e in range(16):
            gvalid = (qoff + e) < n
            cols = jnp.full((16,), colb[e], jnp.int32) + lane16
            rows = jnp.full((16,), e, jnp.int32)
            crow = plsc.load_gather(cbuf, [rows, cols])
            cbs = [jnp.where(gvalid, crow[b], 0.0) for b in range(NB)]
            for k in range(D // 16):
                ev = ebuf[e, pl.ds(k * 16, 16)]
                rv2 = rbuf[e, pl.ds(k * 16, 16)]
                for b in range(NB):
                    s_acc[b, pl.ds(k * 16, 16)] += cbs[b] * ev
                    s_acc[b, pl.ds(D + k * 16, 16)] += cbs[b] * rv2
        return 0

    lax.fori_loop(0, nb, batch_body, 0)

    pltpu.sync_copy(s_acc, part_out.at[wid])
    cntf_v[...] = jnp.full((16,), n.astype(jnp.float32), jnp.float32)
    pltpu.sync_copy(cntf_v, cnt_out.at[wid])


def _tc_body(part_ref, cnt_ref, bases_ref, out_ref):
    s_total = jnp.sum(part_ref[...], axis=0)
    total = jnp.sum(cnt_ref[...]) * (1.0 / 16.0)
    denom = jnp.maximum(total, 1.0)
    acc = jnp.zeros((1, D), jnp.float32)
    for b in range(NB):
        acc = acc + jnp.dot(s_total[b:b + 1, :], bases_ref[b],
                            preferred_element_type=jnp.float32)
    out_ref[...] = acc / denom


def kernel(unseen_entity, triplets, use_cuda, entity_table, relation_table,
           bases, comp):
    trip = jnp.asarray(triplets).astype(jnp.int32)
    pad = jnp.full((TPAD - T,), -1, jnp.int32)
    s_arr = jnp.concatenate([trip[:, 0], pad])
    r_arr = jnp.concatenate([trip[:, 1], jnp.zeros((TPAD - T,), jnp.int32)])
    d_arr = jnp.concatenate([trip[:, 2], pad])
    u_arr = jnp.full((16,), jnp.asarray(unseen_entity, jnp.int32))
    comp_g = comp.astype(jnp.float32).reshape(CG, D)

    part, cnt = _sc_scan(s_arr, r_arr, d_arr, u_arr,
                         entity_table, relation_table, comp_g)

    out = pl.pallas_call(
        _tc_body,
        out_shape=jax.ShapeDtypeStruct((1, D), jnp.float32),
    )(part, cnt, bases)
    return out.reshape(D)
